# Initial kernel scaffold; baseline (speedup 1.0000x reference)
import jax
import jax.numpy as jnp
from jax.experimental import pallas as pl


def kernel(x, edge_attr, We, be, Wn1a, Wn1b, bn1b, bc1, wp1, Wn2a, Wn2b, bn2b, bc2, wp2, Wfc1, bfc1, g1, b1, Wfc2, bfc2, g2, b2, Wfc3, bfc3, edge_index, y):
    raise NotImplementedError("write your pallas kernel here")


# speedup vs baseline: 5.2998x; 5.2998x over previous
import functools

import jax
import jax.numpy as jnp
from jax import lax
from jax.experimental import pallas as pl
from jax.experimental.pallas import tpu as pltpu
from jax.experimental.pallas import tpu_sc as plsc

R = 128
DIM1 = 32
DIM2 = 32
B = 78
N = B * R
E = 319488
K1 = 64
K2 = 32

NW = 32
EPW = E // NW
CHUNK = 128
NCHUNK = EPW // CHUNK
NPT = N // 16


def _wgen_body(wn1a, wn1b, bn1b, wn2a, wn2b, bn2b, w1_o, w2_o):
    w1_o[...] = (jnp.dot(jnp.maximum(wn1a[...], 0.0), wn1b[...],
                         preferred_element_type=jnp.float32)
                 + bn1b[...][None, :])
    w2_o[...] = (jnp.dot(jnp.maximum(wn2a[...], 0.0), wn2b[...],
                         preferred_element_type=jnp.float32)
                 + bn2b[...][None, :])


def _wgen(wn1a, wn1b, bn1b, wn2a, wn2b, bn2b):
    return pl.pallas_call(
        _wgen_body,
        out_shape=[
            jax.ShapeDtypeStruct((R, R * DIM1), jnp.float32),
            jax.ShapeDtypeStruct((R, DIM1 * DIM2), jnp.float32),
        ],
    )(wn1a, wn1b, bn1b, wn2a, wn2b, bn2b)


def _ew_body(we, be, ea, out):
    a = ea[...]
    s = (a[0] * we[0] + a[1] * we[1] + a[2] * we[2] + a[3] * we[3]) + be[0]
    out[...] = jax.nn.sigmoid(s)


def _ew(we, be, ea_t):
    rows = E // 128
    g = 8
    c = rows // g
    return pl.pallas_call(
        _ew_body,
        grid=(g,),
        in_specs=[
            pl.BlockSpec(memory_space=pltpu.SMEM),
            pl.BlockSpec(memory_space=pltpu.SMEM),
            pl.BlockSpec((4, c, 128), lambda i: (0, i, 0)),
        ],
        out_specs=pl.BlockSpec((c, 128), lambda i: (i, 0)),
        out_shape=jax.ShapeDtypeStruct((rows, 128), jnp.float32),
    )(we, be, ea_t)


def _bmm_body(x_ref, w_ref, o_ref):
    o_ref[0] = jnp.dot(x_ref[0], w_ref[0], preferred_element_type=jnp.float32)


def _bmm(xt, w3):
    _, _, din = xt.shape
    dout = w3.shape[2]
    return pl.pallas_call(
        _bmm_body,
        grid=(R,),
        in_specs=[
            pl.BlockSpec((1, B, din), lambda r: (r, 0, 0)),
            pl.BlockSpec((1, din, dout), lambda r: (r, 0, 0)),
        ],
        out_specs=pl.BlockSpec((1, B, dout), lambda r: (r, 0, 0)),
        out_shape=jax.ShapeDtypeStruct((R, B, dout), jnp.float32),
    )(xt, w3)


def _finalize(num, aux, xt, bias):
    den = aux[:, 0]
    noloop = (aux[:, 1] == 0.0).astype(jnp.float32)
    return ((num + noloop[:, None] * xt)
            / jnp.maximum(den + noloop, 1.0)[:, None]
            + bias[None, :])


def _rank_keep(score, elig, k):
    ii = lax.broadcasted_iota(jnp.int32, (R, R), 0)
    jj = lax.broadcasted_iota(jnp.int32, (R, R), 1)
    sj = score[None, :]
    si = score[:, None]
    above = (sj > si) | ((sj == si) & (jj < ii))
    if elig is not None:
        above = above & elig[None, :]
    rank = jnp.sum(above.astype(jnp.int32), axis=1)
    keep = rank < k
    if elig is not None:
        keep = keep & elig
    return keep


def _pool1_body(num0, num1, aux0, aux1, xt, bc1, wp1, keep_o, keepr_o, xp_o,
                x1_o):
    num = num0[0] + num1[0]
    aux = aux0[0] + aux1[0]
    h = _finalize(num, aux, xt[0], bc1[...])
    w = wp1[...]
    nrm = lax.rsqrt(jnp.sum(w * w))
    score = jax.nn.sigmoid(jnp.sum(h * w[None, :], axis=1) * nrm)
    keep = _rank_keep(score, None, K1)
    kf = keep.astype(jnp.float32)
    xp = h * (score * kf)[:, None]
    x1max = jnp.max(jnp.where(keep[:, None], xp, -jnp.inf), axis=0)
    x1mean = jnp.sum(xp, axis=0) * (1.0 / K1)
    keep_o[0, 0] = kf
    keepr_o[0] = kf[:, None] * jnp.ones((1, 16), jnp.float32)
    xp_o[0] = xp
    x1_o[0, 0] = jnp.concatenate([x1max, x1mean])


def _pool1(num0, num1, aux0, aux1, xt, bc1, wp1):
    return pl.pallas_call(
        _pool1_body,
        grid=(B,),
        in_specs=[
            pl.BlockSpec((1, R, DIM1), lambda b: (b, 0, 0)),
            pl.BlockSpec((1, R, DIM1), lambda b: (b, 0, 0)),
            pl.BlockSpec((1, R, 16), lambda b: (b, 0, 0)),
            pl.BlockSpec((1, R, 16), lambda b: (b, 0, 0)),
            pl.BlockSpec((1, R, DIM1), lambda b: (b, 0, 0)),
            pl.BlockSpec((DIM1,), lambda b: (0,)),
            pl.BlockSpec((DIM1,), lambda b: (0,)),
        ],
        out_specs=[
            pl.BlockSpec((1, 1, R), lambda b: (b, 0, 0)),
            pl.BlockSpec((1, R, 16), lambda b: (b, 0, 0)),
            pl.BlockSpec((1, R, DIM1), lambda b: (b, 0, 0)),
            pl.BlockSpec((1, 1, 2 * DIM1), lambda b: (b, 0, 0)),
        ],
        out_shape=[
            jax.ShapeDtypeStruct((B, 1, R), jnp.float32),
            jax.ShapeDtypeStruct((B, R, 16), jnp.float32),
            jax.ShapeDtypeStruct((B, R, DIM1), jnp.float32),
            jax.ShapeDtypeStruct((B, 1, 2 * DIM1), jnp.float32),
        ],
    )(num0, num1, aux0, aux1, xt, bc1, wp1)


def _pool2_body(num0, num1, aux0, aux1, xt, keep1, bc2, wp2, x2_o):
    num = num0[0] + num1[0]
    aux = aux0[0] + aux1[0]
    h = _finalize(num, aux, xt[0], bc2[...])
    w = wp2[...]
    nrm = lax.rsqrt(jnp.sum(w * w))
    score = jax.nn.sigmoid(jnp.sum(h * w[None, :], axis=1) * nrm)
    elig = keep1[0, 0] > 0.5
    keep = _rank_keep(score, elig, K2)
    kf = keep.astype(jnp.float32)
    xq = h * (score * kf)[:, None]
    x2max = jnp.max(jnp.where(keep[:, None], xq, -jnp.inf), axis=0)
    x2mean = jnp.sum(xq, axis=0) * (1.0 / K2)
    x2_o[0, 0] = jnp.concatenate([x2max, x2mean])


def _pool2(num0, num1, aux0, aux1, xt, keep1, bc2, wp2):
    return pl.pallas_call(
        _pool2_body,
        grid=(B,),
        in_specs=[
            pl.BlockSpec((1, R, DIM2), lambda b: (b, 0, 0)),
            pl.BlockSpec((1, R, DIM2), lambda b: (b, 0, 0)),
            pl.BlockSpec((1, R, 16), lambda b: (b, 0, 0)),
            pl.BlockSpec((1, R, 16), lambda b: (b, 0, 0)),
            pl.BlockSpec((1, R, DIM2), lambda b: (b, 0, 0)),
            pl.BlockSpec((1, 1, R), lambda b: (b, 0, 0)),
            pl.BlockSpec((DIM2,), lambda b: (0,)),
            pl.BlockSpec((DIM2,), lambda b: (0,)),
        ],
        out_specs=pl.BlockSpec((1, 1, 2 * DIM2), lambda b: (b, 0, 0)),
        out_shape=jax.ShapeDtypeStruct((B, 1, 2 * DIM2), jnp.float32),
    )(num0, num1, aux0, aux1, xt, keep1, bc2, wp2)


_LN_SCALE = float(1.0 / (1.0 + 1e-5) ** 0.5)


def _head_body(x1, x2, wfc1, bfc1, g1, b1, wfc2, bfc2, g2, b2, wfc3, bfc3,
               out):
    z = jnp.concatenate([x1[...], x2[...]], axis=1)
    z = jnp.maximum(jnp.dot(z, wfc1[...], preferred_element_type=jnp.float32)
                    + bfc1[...][None, :], 0.0)
    z = z * (g1[...] * _LN_SCALE)[None, :] + b1[...][None, :]
    z = jnp.maximum(jnp.dot(z, wfc2[...], preferred_element_type=jnp.float32)
                    + bfc2[...][None, :], 0.0)
    z = z * (g2[...] * _LN_SCALE)[None, :] + b2[...][None, :]
    o = (jnp.dot(z, wfc3[...], preferred_element_type=jnp.float32)
         + bfc3[...][None, :])
    m = jnp.max(o, axis=1, keepdims=True)
    lse = jnp.log(jnp.sum(jnp.exp(o - m), axis=1, keepdims=True)) + m
    out[...] = o - lse


def _head(x1, x2, wfc1, bfc1, g1, b1, wfc2, bfc2, g2, b2, wfc3, bfc3):
    return pl.pallas_call(
        _head_body,
        out_shape=jax.ShapeDtypeStruct((B, 2), jnp.float32),
    )(x1, x2, wfc1, bfc1, g1, b1, wfc2, bfc2, g2, b2, wfc3, bfc3)


_GATHER_DNUMS = lax.GatherDimensionNumbers(
    offset_dims=(), collapsed_slice_dims=(0,), start_index_map=(0,))


def _splat(v, e):
    idx = jnp.full((16, 1), e, jnp.int32)
    return lax.gather(v, idx, _GATHER_DNUMS, (1,),
                      mode=lax.GatherScatterMode.PROMISE_IN_BOUNDS)


def _edge_body(xt_hbm, src_hbm, dst_hbm, ew_hbm, keep_hbm, znum_hbm, zaux_hbm,
               num_out, aux_out,
               num_sh, aux_sh, src_v, dst_v, ew_v,
               rows_v, krs_v, krd_v, msg_v, aux_v):
    c = lax.axis_index("c")
    s = lax.axis_index("s")
    wid = s * 2 + c
    sl = pl.ds(s * NPT, NPT)
    pltpu.sync_copy(znum_hbm.at[sl], num_sh.at[sl])
    pltpu.sync_copy(zaux_hbm.at[sl], aux_sh.at[sl])
    plsc.subcore_barrier()

    base_w = wid * EPW
    lane = lax.iota(jnp.int32, 16)
    is0 = (lane == 0)
    is1 = (lane == 1)
    zeros16 = jnp.zeros((16,), jnp.float32)

    def chunk_body(ci, carry):
        base = base_w + ci * CHUNK
        pltpu.sync_copy(src_hbm.at[pl.ds(base, CHUNK)], src_v)
        pltpu.sync_copy(dst_hbm.at[pl.ds(base, CHUNK)], dst_v)
        pltpu.sync_copy(ew_hbm.at[pl.ds(base, CHUNK)], ew_v)
        pltpu.sync_copy(xt_hbm.at[src_v], rows_v)
        pltpu.sync_copy(keep_hbm.at[src_v], krs_v)
        pltpu.sync_copy(keep_hbm.at[dst_v], krd_v)
        for g in range(CHUNK // 16):
            sv = src_v[pl.ds(g * 16, 16)]
            dv = dst_v[pl.ds(g * 16, 16)]
            ev = ew_v[pl.ds(g * 16, 16)]
            eqf = jnp.where(sv == dv, 1.0, 0.0)
            for e in range(16):
                i = g * 16 + e
                vals = krs_v[i, :] * krd_v[i, :]
                scs = _splat(ev, e) * vals
                lps = _splat(eqf, e) * vals
                msg_v[i, pl.ds(0, 16)] = rows_v[i, pl.ds(0, 16)] * scs
                msg_v[i, pl.ds(16, 16)] = rows_v[i, pl.ds(16, 16)] * scs
                aux_v[i, :] = jnp.where(is0, vals,
                                        jnp.where(is1, lps, zeros16))
        pltpu.sync_copy(msg_v, num_sh.at[dst_v], add=True)
        pltpu.sync_copy(aux_v, aux_sh.at[dst_v], add=True)
        return carry

    lax.fori_loop(0, NCHUNK, chunk_body, 0)
    plsc.subcore_barrier()
    pltpu.sync_copy(num_sh.at[sl], num_out.at[c, sl])
    pltpu.sync_copy(aux_sh.at[sl], aux_out.at[c, sl])


@functools.cache
def _edge_sc():
    return pl.kernel(
        _edge_body,
        out_type=[
            jax.ShapeDtypeStruct((2, N, DIM1), jnp.float32),
            jax.ShapeDtypeStruct((2, N, 16), jnp.float32),
        ],
        mesh=plsc.VectorSubcoreMesh(core_axis_name="c",
                                    subcore_axis_name="s"),
        compiler_params=pltpu.CompilerParams(use_tc_tiling_on_sc=False),
        scratch_types=[
            pltpu.VMEM_SHARED((N, DIM1), jnp.float32),
            pltpu.VMEM_SHARED((N, 16), jnp.float32),
            pltpu.VMEM((CHUNK,), jnp.int32),
            pltpu.VMEM((CHUNK,), jnp.int32),
            pltpu.VMEM((CHUNK,), jnp.float32),
            pltpu.VMEM((CHUNK, DIM1), jnp.float32),
            pltpu.VMEM((CHUNK, 16), jnp.float32),
            pltpu.VMEM((CHUNK, 16), jnp.float32),
            pltpu.VMEM((CHUNK, DIM1), jnp.float32),
            pltpu.VMEM((CHUNK, 16), jnp.float32),
        ],
    )


def _edge_pass(xt, src, dst, ew, keep_rep):
    znum = jnp.zeros((N, DIM1), jnp.float32)
    zaux = jnp.zeros((N, 16), jnp.float32)
    num, aux = _edge_sc()(xt, src, dst, ew, keep_rep, znum, zaux)
    return num, aux


def kernel(x, edge_attr, We, be, Wn1a, Wn1b, bn1b, bc1, wp1, Wn2a, Wn2b,
           bn2b, bc2, wp2, Wfc1, bfc1, g1, b1, Wfc2, bfc2, g2, b2, Wfc3,
           bfc3, edge_index, y):
    src = edge_index[0]
    dst = edge_index[1]

    w1flat, w2flat = _wgen(Wn1a, Wn1b, bn1b, Wn2a, Wn2b, bn2b)
    w1 = w1flat.reshape(R, R, DIM1)
    w2 = w2flat.reshape(R, DIM1, DIM2)

    ea_t = edge_attr.T.reshape(4, E // 128, 128)
    ew = _ew(We.reshape(4), be, ea_t).reshape(E)

    xt1 = (_bmm(x.reshape(B, R, R).transpose(1, 0, 2), w1)
           .transpose(1, 0, 2).reshape(N, DIM1))

    ones = jnp.ones((N, 16), jnp.float32)
    num1, aux1 = _edge_pass(xt1, src, dst, ew, ones)

    xt1g = xt1.reshape(B, R, DIM1)
    keep1, keep1r, xp, x1 = _pool1(num1[0].reshape(B, R, DIM1),
                           num1[1].reshape(B, R, DIM1),
                           aux1[0].reshape(B, R, 16),
                           aux1[1].reshape(B, R, 16),
                           xt1g, bc1, wp1)

    xt2 = (_bmm(xp.transpose(1, 0, 2), w2)
           .transpose(1, 0, 2).reshape(N, DIM2))

    num2, aux2 = _edge_pass(xt2, src, dst, ew, keep1r.reshape(N, 16))

    x2 = _pool2(num2[0].reshape(B, R, DIM2),
                num2[1].reshape(B, R, DIM2),
                aux2[0].reshape(B, R, 16),
                aux2[1].reshape(B, R, 16),
                xt2.reshape(B, R, DIM2), keep1, bc2, wp2)

    return _head(x1.reshape(B, 2 * DIM1), x2.reshape(B, 2 * DIM2),
                 Wfc1, bfc1, g1, b1, Wfc2, bfc2, g2, b2, Wfc3, bfc3)


# speedup vs baseline: 6.1884x; 1.1677x over previous
import functools

import jax
import jax.numpy as jnp
from jax import lax
from jax.experimental import pallas as pl
from jax.experimental.pallas import tpu as pltpu
from jax.experimental.pallas import tpu_sc as plsc

R = 128
DIM1 = 32
DIM2 = 32
B = 78
N = B * R
E = 319488
K1 = 64
K2 = 32

NW = 32
EPW = E // NW
CHUNK = 128
NCHUNK = EPW // CHUNK
NPT = N // 16


def _wgen_body(wn1a, wn1b, bn1b, wn2a, wn2b, bn2b, w1_o, w2_o):
    w1_o[...] = (jnp.dot(jnp.maximum(wn1a[...], 0.0), wn1b[...],
                         preferred_element_type=jnp.float32)
                 + bn1b[...][None, :])
    w2_o[...] = (jnp.dot(jnp.maximum(wn2a[...], 0.0), wn2b[...],
                         preferred_element_type=jnp.float32)
                 + bn2b[...][None, :])


def _wgen(wn1a, wn1b, bn1b, wn2a, wn2b, bn2b):
    return pl.pallas_call(
        _wgen_body,
        out_shape=[
            jax.ShapeDtypeStruct((R, R * DIM1), jnp.float32),
            jax.ShapeDtypeStruct((R, DIM1 * DIM2), jnp.float32),
        ],
    )(wn1a, wn1b, bn1b, wn2a, wn2b, bn2b)


def _ew_body(we, be, ea, out):
    a = ea[...]
    s = (a[0] * we[0] + a[1] * we[1] + a[2] * we[2] + a[3] * we[3]) + be[0]
    out[...] = jax.nn.sigmoid(s)


def _ew(we, be, ea_t):
    rows = E // 128
    g = 8
    c = rows // g
    return pl.pallas_call(
        _ew_body,
        grid=(g,),
        in_specs=[
            pl.BlockSpec(memory_space=pltpu.SMEM),
            pl.BlockSpec(memory_space=pltpu.SMEM),
            pl.BlockSpec((4, c, 128), lambda i: (0, i, 0)),
        ],
        out_specs=pl.BlockSpec((c, 128), lambda i: (i, 0)),
        out_shape=jax.ShapeDtypeStruct((rows, 128), jnp.float32),
    )(we, be, ea_t)


def _bmm_body(x_ref, w_ref, o_ref):
    o_ref[0] = jnp.dot(x_ref[0], w_ref[0], preferred_element_type=jnp.float32)


def _bmm(xt, w3):
    _, _, din = xt.shape
    dout = w3.shape[2]
    return pl.pallas_call(
        _bmm_body,
        grid=(R,),
        in_specs=[
            pl.BlockSpec((1, B, din), lambda r: (r, 0, 0)),
            pl.BlockSpec((1, din, dout), lambda r: (r, 0, 0)),
        ],
        out_specs=pl.BlockSpec((1, B, dout), lambda r: (r, 0, 0)),
        out_shape=jax.ShapeDtypeStruct((R, B, dout), jnp.float32),
    )(xt, w3)


def _finalize(num, aux, xt, bias):
    den = aux[:, 0]
    noloop = (aux[:, 1] == 0.0).astype(jnp.float32)
    return ((num + noloop[:, None] * xt)
            / jnp.maximum(den + noloop, 1.0)[:, None]
            + bias[None, :])


def _rank_keep(score, elig, k):
    ii = lax.broadcasted_iota(jnp.int32, (R, R), 0)
    jj = lax.broadcasted_iota(jnp.int32, (R, R), 1)
    sj = score[None, :]
    si = score[:, None]
    above = (sj > si) | ((sj == si) & (jj < ii))
    if elig is not None:
        above = above & elig[None, :]
    rank = jnp.sum(above.astype(jnp.int32), axis=1)
    keep = rank < k
    if elig is not None:
        keep = keep & elig
    return keep


def _pool1_body(num0, num1, aux0, aux1, xt, bc1, wp1, keep_o, keepr_o, xp_o,
                x1_o):
    num = num0[0] + num1[0]
    aux = aux0[0] + aux1[0]
    h = _finalize(num, aux, xt[0], bc1[...])
    w = wp1[...]
    nrm = lax.rsqrt(jnp.sum(w * w))
    score = jax.nn.sigmoid(jnp.sum(h * w[None, :], axis=1) * nrm)
    keep = _rank_keep(score, None, K1)
    kf = keep.astype(jnp.float32)
    xp = h * (score * kf)[:, None]
    x1max = jnp.max(jnp.where(keep[:, None], xp, -jnp.inf), axis=0)
    x1mean = jnp.sum(xp, axis=0) * (1.0 / K1)
    keep_o[0, 0] = kf
    keepr_o[0] = kf[:, None] * jnp.ones((1, 16), jnp.float32)
    xp_o[0] = xp
    x1_o[0, 0] = jnp.concatenate([x1max, x1mean])


def _pool1(num0, num1, aux0, aux1, xt, bc1, wp1):
    return pl.pallas_call(
        _pool1_body,
        grid=(B,),
        in_specs=[
            pl.BlockSpec((1, R, DIM1), lambda b: (b, 0, 0)),
            pl.BlockSpec((1, R, DIM1), lambda b: (b, 0, 0)),
            pl.BlockSpec((1, R, 16), lambda b: (b, 0, 0)),
            pl.BlockSpec((1, R, 16), lambda b: (b, 0, 0)),
            pl.BlockSpec((1, R, DIM1), lambda b: (b, 0, 0)),
            pl.BlockSpec((DIM1,), lambda b: (0,)),
            pl.BlockSpec((DIM1,), lambda b: (0,)),
        ],
        out_specs=[
            pl.BlockSpec((1, 1, R), lambda b: (b, 0, 0)),
            pl.BlockSpec((1, R, 16), lambda b: (b, 0, 0)),
            pl.BlockSpec((1, R, DIM1), lambda b: (b, 0, 0)),
            pl.BlockSpec((1, 1, 2 * DIM1), lambda b: (b, 0, 0)),
        ],
        out_shape=[
            jax.ShapeDtypeStruct((B, 1, R), jnp.float32),
            jax.ShapeDtypeStruct((B, R, 16), jnp.float32),
            jax.ShapeDtypeStruct((B, R, DIM1), jnp.float32),
            jax.ShapeDtypeStruct((B, 1, 2 * DIM1), jnp.float32),
        ],
    )(num0, num1, aux0, aux1, xt, bc1, wp1)


def _pool2_body(num0, num1, aux0, aux1, xt, keep1, bc2, wp2, x2_o):
    num = num0[0] + num1[0]
    aux = aux0[0] + aux1[0]
    h = _finalize(num, aux, xt[0], bc2[...])
    w = wp2[...]
    nrm = lax.rsqrt(jnp.sum(w * w))
    score = jax.nn.sigmoid(jnp.sum(h * w[None, :], axis=1) * nrm)
    elig = keep1[0, 0] > 0.5
    keep = _rank_keep(score, elig, K2)
    kf = keep.astype(jnp.float32)
    xq = h * (score * kf)[:, None]
    x2max = jnp.max(jnp.where(keep[:, None], xq, -jnp.inf), axis=0)
    x2mean = jnp.sum(xq, axis=0) * (1.0 / K2)
    x2_o[0, 0] = jnp.concatenate([x2max, x2mean])


def _pool2(num0, num1, aux0, aux1, xt, keep1, bc2, wp2):
    return pl.pallas_call(
        _pool2_body,
        grid=(B,),
        in_specs=[
            pl.BlockSpec((1, R, DIM2), lambda b: (b, 0, 0)),
            pl.BlockSpec((1, R, DIM2), lambda b: (b, 0, 0)),
            pl.BlockSpec((1, R, 16), lambda b: (b, 0, 0)),
            pl.BlockSpec((1, R, 16), lambda b: (b, 0, 0)),
            pl.BlockSpec((1, R, DIM2), lambda b: (b, 0, 0)),
            pl.BlockSpec((1, 1, R), lambda b: (b, 0, 0)),
            pl.BlockSpec((DIM2,), lambda b: (0,)),
            pl.BlockSpec((DIM2,), lambda b: (0,)),
        ],
        out_specs=pl.BlockSpec((1, 1, 2 * DIM2), lambda b: (b, 0, 0)),
        out_shape=jax.ShapeDtypeStruct((B, 1, 2 * DIM2), jnp.float32),
    )(num0, num1, aux0, aux1, xt, keep1, bc2, wp2)


_LN_SCALE = float(1.0 / (1.0 + 1e-5) ** 0.5)


def _head_body(x1, x2, wfc1, bfc1, g1, b1, wfc2, bfc2, g2, b2, wfc3, bfc3,
               out):
    z = jnp.concatenate([x1[...], x2[...]], axis=1)
    z = jnp.maximum(jnp.dot(z, wfc1[...], preferred_element_type=jnp.float32)
                    + bfc1[...][None, :], 0.0)
    z = z * (g1[...] * _LN_SCALE)[None, :] + b1[...][None, :]
    z = jnp.maximum(jnp.dot(z, wfc2[...], preferred_element_type=jnp.float32)
                    + bfc2[...][None, :], 0.0)
    z = z * (g2[...] * _LN_SCALE)[None, :] + b2[...][None, :]
    o = (jnp.dot(z, wfc3[...], preferred_element_type=jnp.float32)
         + bfc3[...][None, :])
    m = jnp.max(o, axis=1, keepdims=True)
    lse = jnp.log(jnp.sum(jnp.exp(o - m), axis=1, keepdims=True)) + m
    out[...] = o - lse


def _head(x1, x2, wfc1, bfc1, g1, b1, wfc2, bfc2, g2, b2, wfc3, bfc3):
    return pl.pallas_call(
        _head_body,
        out_shape=jax.ShapeDtypeStruct((B, 2), jnp.float32),
    )(x1, x2, wfc1, bfc1, g1, b1, wfc2, bfc2, g2, b2, wfc3, bfc3)


_GATHER_DNUMS = lax.GatherDimensionNumbers(
    offset_dims=(), collapsed_slice_dims=(0,), start_index_map=(0,))


def _splat(v, e):
    idx = jnp.full((16, 1), e, jnp.int32)
    return lax.gather(v, idx, _GATHER_DNUMS, (1,),
                      mode=lax.GatherScatterMode.PROMISE_IN_BOUNDS)


def _edge_body(xt_hbm, epk_hbm, ew_hbm, keep_hbm, znum_hbm, zaux_hbm,
               num_out, aux_out,
               num_sh, aux_sh, pk_v, ew_v, src_v, dst_v,
               rows_v, krs_v, krd_v, msg_v, aux_v, gsem, ssem):
    c = lax.axis_index("c")
    s = lax.axis_index("s")
    wid = s * 2 + c
    sl = pl.ds(s * NPT, NPT)
    pltpu.sync_copy(znum_hbm.at[sl], num_sh.at[sl])
    pltpu.sync_copy(zaux_hbm.at[sl], aux_sh.at[sl])
    plsc.subcore_barrier()

    base_w = wid * EPW
    lane = lax.iota(jnp.int32, 16)
    is0 = (lane == 0)
    is1 = (lane == 1)
    zeros16 = jnp.zeros((16,), jnp.float32)

    def chunk_body(ci, carry):
        base = base_w + ci * CHUNK
        pltpu.sync_copy(epk_hbm.at[:, pl.ds(base, CHUNK)], pk_v)
        pltpu.sync_copy(ew_hbm.at[pl.ds(base, CHUNK)], ew_v)
        src_v[...] = pk_v[0, :]
        dst_v[...] = pk_v[1, :]
        g1 = pltpu.async_copy(xt_hbm.at[src_v], rows_v, gsem)
        g2 = pltpu.async_copy(keep_hbm.at[src_v], krs_v, gsem)
        g3 = pltpu.async_copy(keep_hbm.at[dst_v], krd_v, gsem)
        g1.wait()
        g2.wait()
        g3.wait()
        for g in range(CHUNK // 16):
            sv = pk_v[0, pl.ds(g * 16, 16)]
            dv = pk_v[1, pl.ds(g * 16, 16)]
            ev = ew_v[pl.ds(g * 16, 16)]
            eqf = jnp.where(sv == dv, 1.0, 0.0)
            for e in range(16):
                i = g * 16 + e
                vals = krs_v[i, :] * krd_v[i, :]
                scs = _splat(ev, e) * vals
                lps = _splat(eqf, e) * vals
                msg_v[i, pl.ds(0, 16)] = rows_v[i, pl.ds(0, 16)] * scs
                msg_v[i, pl.ds(16, 16)] = rows_v[i, pl.ds(16, 16)] * scs
                aux_v[i, :] = jnp.where(is0, vals,
                                        jnp.where(is1, lps, zeros16))
        s1 = pltpu.async_copy(msg_v, num_sh.at[dst_v], ssem, add=True)
        s2 = pltpu.async_copy(aux_v, aux_sh.at[dst_v], ssem, add=True)
        s1.wait()
        s2.wait()
        return carry

    lax.fori_loop(0, NCHUNK, chunk_body, 0)
    plsc.subcore_barrier()
    pltpu.sync_copy(num_sh.at[sl], num_out.at[c, sl])
    pltpu.sync_copy(aux_sh.at[sl], aux_out.at[c, sl])


@functools.cache
def _edge_sc():
    return pl.kernel(
        _edge_body,
        out_type=[
            jax.ShapeDtypeStruct((2, N, DIM1), jnp.float32),
            jax.ShapeDtypeStruct((2, N, 16), jnp.float32),
        ],
        mesh=plsc.VectorSubcoreMesh(core_axis_name="c",
                                    subcore_axis_name="s"),
        compiler_params=pltpu.CompilerParams(use_tc_tiling_on_sc=False),
        scratch_types=[
            pltpu.VMEM_SHARED((N, DIM1), jnp.float32),
            pltpu.VMEM_SHARED((N, 16), jnp.float32),
            pltpu.VMEM((2, CHUNK), jnp.int32),
            pltpu.VMEM((CHUNK,), jnp.float32),
            pltpu.VMEM((CHUNK,), jnp.int32),
            pltpu.VMEM((CHUNK,), jnp.int32),
            pltpu.VMEM((CHUNK, DIM1), jnp.float32),
            pltpu.VMEM((CHUNK, 16), jnp.float32),
            pltpu.VMEM((CHUNK, 16), jnp.float32),
            pltpu.VMEM((CHUNK, DIM1), jnp.float32),
            pltpu.VMEM((CHUNK, 16), jnp.float32),
            pltpu.SemaphoreType.DMA,
            pltpu.SemaphoreType.DMA,
        ],
    )


def _edge_pass(xt, edge_pack, ew, keep_rep):
    znum = jnp.zeros((N, DIM1), jnp.float32)
    zaux = jnp.zeros((N, 16), jnp.float32)
    num, aux = _edge_sc()(xt, edge_pack, ew, keep_rep, znum, zaux)
    return num, aux


def kernel(x, edge_attr, We, be, Wn1a, Wn1b, bn1b, bc1, wp1, Wn2a, Wn2b,
           bn2b, bc2, wp2, Wfc1, bfc1, g1, b1, Wfc2, bfc2, g2, b2, Wfc3,
           bfc3, edge_index, y):
    edge_pack = edge_index

    w1flat, w2flat = _wgen(Wn1a, Wn1b, bn1b, Wn2a, Wn2b, bn2b)
    w1 = w1flat.reshape(R, R, DIM1)
    w2 = w2flat.reshape(R, DIM1, DIM2)

    ea_t = edge_attr.T.reshape(4, E // 128, 128)
    ew = _ew(We.reshape(4), be, ea_t).reshape(E)

    xt1 = (_bmm(x.reshape(B, R, R).transpose(1, 0, 2), w1)
           .transpose(1, 0, 2).reshape(N, DIM1))

    ones = jnp.ones((N, 16), jnp.float32)
    num1, aux1 = _edge_pass(xt1, edge_pack, ew, ones)

    xt1g = xt1.reshape(B, R, DIM1)
    keep1, keep1r, xp, x1 = _pool1(num1[0].reshape(B, R, DIM1),
                           num1[1].reshape(B, R, DIM1),
                           aux1[0].reshape(B, R, 16),
                           aux1[1].reshape(B, R, 16),
                           xt1g, bc1, wp1)

    xt2 = (_bmm(xp.transpose(1, 0, 2), w2)
           .transpose(1, 0, 2).reshape(N, DIM2))

    num2, aux2 = _edge_pass(xt2, edge_pack, ew, keep1r.reshape(N, 16))

    x2 = _pool2(num2[0].reshape(B, R, DIM2),
                num2[1].reshape(B, R, DIM2),
                aux2[0].reshape(B, R, 16),
                aux2[1].reshape(B, R, 16),
                xt2.reshape(B, R, DIM2), keep1, bc2, wp2)

    return _head(x1.reshape(B, 2 * DIM1), x2.reshape(B, 2 * DIM2),
                 Wfc1, bfc1, g1, b1, Wfc2, bfc2, g2, b2, Wfc3, bfc3)


# speedup vs baseline: 8.7461x; 1.4133x over previous
import functools

import jax
import jax.numpy as jnp
from jax import lax
from jax.experimental import pallas as pl
from jax.experimental.pallas import tpu as pltpu
from jax.experimental.pallas import tpu_sc as plsc

R = 128
DIM1 = 32
DIM2 = 32
B = 78
N = B * R
E = 319488
K1 = 64
K2 = 32

NW = 32
EPW = E // NW
CHUNK = 128
NCHUNK = EPW // CHUNK
NPT = N // 16


def _wgen_body(wn1a, wn1b, bn1b, wn2a, wn2b, bn2b, w1_o, w2_o):
    w1_o[...] = (jnp.dot(jnp.maximum(wn1a[...], 0.0), wn1b[...],
                         preferred_element_type=jnp.float32)
                 + bn1b[...][None, :])
    w2_o[...] = (jnp.dot(jnp.maximum(wn2a[...], 0.0), wn2b[...],
                         preferred_element_type=jnp.float32)
                 + bn2b[...][None, :])


def _wgen(wn1a, wn1b, bn1b, wn2a, wn2b, bn2b):
    return pl.pallas_call(
        _wgen_body,
        out_shape=[
            jax.ShapeDtypeStruct((R, R * DIM1), jnp.float32),
            jax.ShapeDtypeStruct((R, DIM1 * DIM2), jnp.float32),
        ],
    )(wn1a, wn1b, bn1b, wn2a, wn2b, bn2b)


def _ew_body(we, be, ea, out):
    a = ea[...]
    s = (a[0] * we[0] + a[1] * we[1] + a[2] * we[2] + a[3] * we[3]) + be[0]
    out[...] = jax.nn.sigmoid(s)


def _ew(we, be, ea_t):
    rows = E // 128
    g = 8
    c = rows // g
    return pl.pallas_call(
        _ew_body,
        grid=(g,),
        in_specs=[
            pl.BlockSpec(memory_space=pltpu.SMEM),
            pl.BlockSpec(memory_space=pltpu.SMEM),
            pl.BlockSpec((4, c, 128), lambda i: (0, i, 0)),
        ],
        out_specs=pl.BlockSpec((c, 128), lambda i: (i, 0)),
        out_shape=jax.ShapeDtypeStruct((rows, 128), jnp.float32),
    )(we, be, ea_t)


_RB = 8


def _bmm_body(x_ref, w_ref, o_ref):
    for r in range(_RB):
        o_ref[r] = jnp.dot(x_ref[r], w_ref[r],
                           preferred_element_type=jnp.float32)


def _bmm(xt, w3):
    _, _, din = xt.shape
    dout = w3.shape[2]
    return pl.pallas_call(
        _bmm_body,
        grid=(R // _RB,),
        in_specs=[
            pl.BlockSpec((_RB, B, din), lambda r: (r, 0, 0)),
            pl.BlockSpec((_RB, din, dout), lambda r: (r, 0, 0)),
        ],
        out_specs=pl.BlockSpec((_RB, B, dout), lambda r: (r, 0, 0)),
        out_shape=jax.ShapeDtypeStruct((R, B, dout), jnp.float32),
    )(xt, w3)


def _finalize(num, aux, xt, bias):
    den = aux[:, :, 0]
    noloop = (aux[:, :, 1] == 0.0).astype(jnp.float32)
    return ((num + noloop[:, :, None] * xt)
            / jnp.maximum(den + noloop, 1.0)[:, :, None]
            + bias[None, None, :])


_GB = 6


def _rank_keep(score, elig, k):
    ii = lax.broadcasted_iota(jnp.int32, (_GB, R, R), 1)
    jj = lax.broadcasted_iota(jnp.int32, (_GB, R, R), 2)
    sj = score[:, None, :]
    si = score[:, :, None]
    above = (sj > si) | ((sj == si) & (jj < ii))
    if elig is not None:
        above = above & elig[:, None, :]
    rank = jnp.sum(above.astype(jnp.int32), axis=2)
    keep = rank < k
    if elig is not None:
        keep = keep & elig
    return keep


def _pool1_body(num0, num1, aux0, aux1, xt, bc1, wp1, keep_o, keepr_o, xp_o,
                x1_o):
    num = num0[...] + num1[...]
    aux = aux0[...] + aux1[...]
    h = _finalize(num, aux, xt[...], bc1[...])
    w = wp1[...]
    nrm = lax.rsqrt(jnp.sum(w * w))
    score = jax.nn.sigmoid(jnp.sum(h * w[None, None, :], axis=2) * nrm)
    keep = _rank_keep(score, None, K1)
    kf = keep.astype(jnp.float32)
    xp = h * (score * kf)[:, :, None]
    x1max = jnp.max(jnp.where(keep[:, :, None], xp, -jnp.inf), axis=1)
    x1mean = jnp.sum(xp, axis=1) * (1.0 / K1)
    keep_o[...] = kf[:, None, :]
    keepr_o[...] = kf[:, :, None] * jnp.ones((1, 1, 16), jnp.float32)
    xp_o[...] = xp
    x1_o[...] = jnp.concatenate([x1max, x1mean], axis=1)[:, None, :]


def _pool1(num0, num1, aux0, aux1, xt, bc1, wp1):
    return pl.pallas_call(
        _pool1_body,
        grid=(B // _GB,),
        in_specs=[
            pl.BlockSpec((_GB, R, DIM1), lambda b: (b, 0, 0)),
            pl.BlockSpec((_GB, R, DIM1), lambda b: (b, 0, 0)),
            pl.BlockSpec((_GB, R, 16), lambda b: (b, 0, 0)),
            pl.BlockSpec((_GB, R, 16), lambda b: (b, 0, 0)),
            pl.BlockSpec((_GB, R, DIM1), lambda b: (b, 0, 0)),
            pl.BlockSpec((DIM1,), lambda b: (0,)),
            pl.BlockSpec((DIM1,), lambda b: (0,)),
        ],
        out_specs=[
            pl.BlockSpec((_GB, 1, R), lambda b: (b, 0, 0)),
            pl.BlockSpec((_GB, R, 16), lambda b: (b, 0, 0)),
            pl.BlockSpec((_GB, R, DIM1), lambda b: (b, 0, 0)),
            pl.BlockSpec((_GB, 1, 2 * DIM1), lambda b: (b, 0, 0)),
        ],
        out_shape=[
            jax.ShapeDtypeStruct((B, 1, R), jnp.float32),
            jax.ShapeDtypeStruct((B, R, 16), jnp.float32),
            jax.ShapeDtypeStruct((B, R, DIM1), jnp.float32),
            jax.ShapeDtypeStruct((B, 1, 2 * DIM1), jnp.float32),
        ],
    )(num0, num1, aux0, aux1, xt, bc1, wp1)


def _pool2_body(num0, num1, aux0, aux1, xt, keep1, bc2, wp2, x2_o):
    num = num0[...] + num1[...]
    aux = aux0[...] + aux1[...]
    h = _finalize(num, aux, xt[...], bc2[...])
    w = wp2[...]
    nrm = lax.rsqrt(jnp.sum(w * w))
    score = jax.nn.sigmoid(jnp.sum(h * w[None, None, :], axis=2) * nrm)
    elig = keep1[:, 0, :] > 0.5
    keep = _rank_keep(score, elig, K2)
    kf = keep.astype(jnp.float32)
    xq = h * (score * kf)[:, :, None]
    x2max = jnp.max(jnp.where(keep[:, :, None], xq, -jnp.inf), axis=1)
    x2mean = jnp.sum(xq, axis=1) * (1.0 / K2)
    x2_o[...] = jnp.concatenate([x2max, x2mean], axis=1)[:, None, :]


def _pool2(num0, num1, aux0, aux1, xt, keep1, bc2, wp2):
    return pl.pallas_call(
        _pool2_body,
        grid=(B // _GB,),
        in_specs=[
            pl.BlockSpec((_GB, R, DIM2), lambda b: (b, 0, 0)),
            pl.BlockSpec((_GB, R, DIM2), lambda b: (b, 0, 0)),
            pl.BlockSpec((_GB, R, 16), lambda b: (b, 0, 0)),
            pl.BlockSpec((_GB, R, 16), lambda b: (b, 0, 0)),
            pl.BlockSpec((_GB, R, DIM2), lambda b: (b, 0, 0)),
            pl.BlockSpec((_GB, 1, R), lambda b: (b, 0, 0)),
            pl.BlockSpec((DIM2,), lambda b: (0,)),
            pl.BlockSpec((DIM2,), lambda b: (0,)),
        ],
        out_specs=pl.BlockSpec((_GB, 1, 2 * DIM2), lambda b: (b, 0, 0)),
        out_shape=jax.ShapeDtypeStruct((B, 1, 2 * DIM2), jnp.float32),
    )(num0, num1, aux0, aux1, xt, keep1, bc2, wp2)


_LN_SCALE = float(1.0 / (1.0 + 1e-5) ** 0.5)


def _head_body(x1, x2, wfc1, bfc1, g1, b1, wfc2, bfc2, g2, b2, wfc3, bfc3,
               out):
    z = jnp.concatenate([x1[...], x2[...]], axis=1)
    z = jnp.maximum(jnp.dot(z, wfc1[...], preferred_element_type=jnp.float32)
                    + bfc1[...][None, :], 0.0)
    z = z * (g1[...] * _LN_SCALE)[None, :] + b1[...][None, :]
    z = jnp.maximum(jnp.dot(z, wfc2[...], preferred_element_type=jnp.float32)
                    + bfc2[...][None, :], 0.0)
    z = z * (g2[...] * _LN_SCALE)[None, :] + b2[...][None, :]
    o = (jnp.dot(z, wfc3[...], preferred_element_type=jnp.float32)
         + bfc3[...][None, :])
    m = jnp.max(o, axis=1, keepdims=True)
    lse = jnp.log(jnp.sum(jnp.exp(o - m), axis=1, keepdims=True)) + m
    out[...] = o - lse


def _head(x1, x2, wfc1, bfc1, g1, b1, wfc2, bfc2, g2, b2, wfc3, bfc3):
    return pl.pallas_call(
        _head_body,
        out_shape=jax.ShapeDtypeStruct((B, 2), jnp.float32),
    )(x1, x2, wfc1, bfc1, g1, b1, wfc2, bfc2, g2, b2, wfc3, bfc3)


_GATHER_DNUMS = lax.GatherDimensionNumbers(
    offset_dims=(), collapsed_slice_dims=(0,), start_index_map=(0,))


def _splat(v, e):
    idx = jnp.full((16, 1), e, jnp.int32)
    return lax.gather(v, idx, _GATHER_DNUMS, (1,),
                      mode=lax.GatherScatterMode.PROMISE_IN_BOUNDS)


def _edge_body(xt_hbm, epk_hbm, ew_hbm, keep_hbm, znum_hbm, zaux_hbm,
               num_out, aux_out,
               num_sh, aux_sh, pk_v, ew_v, src_v, dst_v,
               rows_v, krs_v, krd_v, msg_v, aux_v, gsem, ssem):
    c = lax.axis_index("c")
    s = lax.axis_index("s")
    wid = s * 2 + c
    sl = pl.ds(s * NPT, NPT)
    pltpu.sync_copy(znum_hbm.at[sl], num_sh.at[sl])
    pltpu.sync_copy(zaux_hbm.at[sl], aux_sh.at[sl])
    plsc.subcore_barrier()

    base_w = wid * EPW
    lane = lax.iota(jnp.int32, 16)
    is0 = (lane == 0)
    is1 = (lane == 1)
    zeros16 = jnp.zeros((16,), jnp.float32)

    def chunk_body(ci, carry):
        base = base_w + ci * CHUNK
        pltpu.sync_copy(epk_hbm.at[:, pl.ds(base, CHUNK)], pk_v)
        pltpu.sync_copy(ew_hbm.at[pl.ds(base, CHUNK)], ew_v)
        src_v[...] = pk_v[0, :]
        dst_v[...] = pk_v[1, :]
        g1 = pltpu.async_copy(xt_hbm.at[src_v], rows_v, gsem)
        g2 = pltpu.async_copy(keep_hbm.at[src_v], krs_v, gsem)
        g3 = pltpu.async_copy(keep_hbm.at[dst_v], krd_v, gsem)
        g1.wait()
        g2.wait()
        g3.wait()
        for g in range(CHUNK // 16):
            sv = pk_v[0, pl.ds(g * 16, 16)]
            dv = pk_v[1, pl.ds(g * 16, 16)]
            ev = ew_v[pl.ds(g * 16, 16)]
            eqf = jnp.where(sv == dv, 1.0, 0.0)
            for e in range(16):
                i = g * 16 + e
                vals = krs_v[i, :] * krd_v[i, :]
                scs = _splat(ev, e) * vals
                lps = _splat(eqf, e) * vals
                msg_v[i, pl.ds(0, 16)] = rows_v[i, pl.ds(0, 16)] * scs
                msg_v[i, pl.ds(16, 16)] = rows_v[i, pl.ds(16, 16)] * scs
                aux_v[i, :] = jnp.where(is0, vals,
                                        jnp.where(is1, lps, zeros16))
        s1 = pltpu.async_copy(msg_v, num_sh.at[dst_v], ssem, add=True)
        s2 = pltpu.async_copy(aux_v, aux_sh.at[dst_v], ssem, add=True)
        s1.wait()
        s2.wait()
        return carry

    lax.fori_loop(0, NCHUNK, chunk_body, 0)
    plsc.subcore_barrier()
    pltpu.sync_copy(num_sh.at[sl], num_out.at[c, sl])
    pltpu.sync_copy(aux_sh.at[sl], aux_out.at[c, sl])


@functools.cache
def _edge_sc():
    return pl.kernel(
        _edge_body,
        out_type=[
            jax.ShapeDtypeStruct((2, N, DIM1), jnp.float32),
            jax.ShapeDtypeStruct((2, N, 16), jnp.float32),
        ],
        mesh=plsc.VectorSubcoreMesh(core_axis_name="c",
                                    subcore_axis_name="s"),
        compiler_params=pltpu.CompilerParams(use_tc_tiling_on_sc=False),
        scratch_types=[
            pltpu.VMEM_SHARED((N, DIM1), jnp.float32),
            pltpu.VMEM_SHARED((N, 16), jnp.float32),
            pltpu.VMEM((2, CHUNK), jnp.int32),
            pltpu.VMEM((CHUNK,), jnp.float32),
            pltpu.VMEM((CHUNK,), jnp.int32),
            pltpu.VMEM((CHUNK,), jnp.int32),
            pltpu.VMEM((CHUNK, DIM1), jnp.float32),
            pltpu.VMEM((CHUNK, 16), jnp.float32),
            pltpu.VMEM((CHUNK, 16), jnp.float32),
            pltpu.VMEM((CHUNK, DIM1), jnp.float32),
            pltpu.VMEM((CHUNK, 16), jnp.float32),
            pltpu.SemaphoreType.DMA,
            pltpu.SemaphoreType.DMA,
        ],
    )


def _edge_pass(xt, edge_pack, ew, keep_rep):
    znum = jnp.zeros((N, DIM1), jnp.float32)
    zaux = jnp.zeros((N, 16), jnp.float32)
    num, aux = _edge_sc()(xt, edge_pack, ew, keep_rep, znum, zaux)
    return num, aux


def kernel(x, edge_attr, We, be, Wn1a, Wn1b, bn1b, bc1, wp1, Wn2a, Wn2b,
           bn2b, bc2, wp2, Wfc1, bfc1, g1, b1, Wfc2, bfc2, g2, b2, Wfc3,
           bfc3, edge_index, y):
    edge_pack = edge_index

    w1flat, w2flat = _wgen(Wn1a, Wn1b, bn1b, Wn2a, Wn2b, bn2b)
    w1 = w1flat.reshape(R, R, DIM1)
    w2 = w2flat.reshape(R, DIM1, DIM2)

    ea_t = edge_attr.T.reshape(4, E // 128, 128)
    ew = _ew(We.reshape(4), be, ea_t).reshape(E)

    xt1 = (_bmm(x.reshape(B, R, R).transpose(1, 0, 2), w1)
           .transpose(1, 0, 2).reshape(N, DIM1))

    ones = jnp.ones((N, 16), jnp.float32)
    num1, aux1 = _edge_pass(xt1, edge_pack, ew, ones)

    xt1g = xt1.reshape(B, R, DIM1)
    keep1, keep1r, xp, x1 = _pool1(num1[0].reshape(B, R, DIM1),
                           num1[1].reshape(B, R, DIM1),
                           aux1[0].reshape(B, R, 16),
                           aux1[1].reshape(B, R, 16),
                           xt1g, bc1, wp1)

    xt2 = (_bmm(xp.transpose(1, 0, 2), w2)
           .transpose(1, 0, 2).reshape(N, DIM2))

    num2, aux2 = _edge_pass(xt2, edge_pack, ew, keep1r.reshape(N, 16))

    x2 = _pool2(num2[0].reshape(B, R, DIM2),
                num2[1].reshape(B, R, DIM2),
                aux2[0].reshape(B, R, 16),
                aux2[1].reshape(B, R, 16),
                xt2.reshape(B, R, DIM2), keep1, bc2, wp2)

    return _head(x1.reshape(B, 2 * DIM1), x2.reshape(B, 2 * DIM2),
                 Wfc1, bfc1, g1, b1, Wfc2, bfc2, g2, b2, Wfc3, bfc3)


# speedup vs baseline: 10.4754x; 1.1977x over previous
import functools

import jax
import jax.numpy as jnp
from jax import lax
from jax.experimental import pallas as pl
from jax.experimental.pallas import tpu as pltpu
from jax.experimental.pallas import tpu_sc as plsc

R = 128
DIM1 = 32
DIM2 = 32
B = 78
N = B * R
E = 319488
K1 = 64
K2 = 32

NW = 32
EPW = E // NW
CHUNK = 128
NCHUNK = EPW // CHUNK
NPT = N // 16


def _wgen_body(wn1a, wn1b, bn1b, wn2a, wn2b, bn2b, w1_o, w2_o):
    w1_o[...] = (jnp.dot(jnp.maximum(wn1a[...], 0.0), wn1b[...],
                         preferred_element_type=jnp.float32)
                 + bn1b[...][None, :])
    w2_o[...] = (jnp.dot(jnp.maximum(wn2a[...], 0.0), wn2b[...],
                         preferred_element_type=jnp.float32)
                 + bn2b[...][None, :])


def _wgen(wn1a, wn1b, bn1b, wn2a, wn2b, bn2b):
    return pl.pallas_call(
        _wgen_body,
        out_shape=[
            jax.ShapeDtypeStruct((R, R * DIM1), jnp.float32),
            jax.ShapeDtypeStruct((R, DIM1 * DIM2), jnp.float32),
        ],
    )(wn1a, wn1b, bn1b, wn2a, wn2b, bn2b)


def _ew_body(we, be, ea, out):
    a = ea[...]
    s = (a[0] * we[0] + a[1] * we[1] + a[2] * we[2] + a[3] * we[3]) + be[0]
    out[...] = jax.nn.sigmoid(s)


def _ew(we, be, ea_t):
    rows = E // 128
    g = 8
    c = rows // g
    return pl.pallas_call(
        _ew_body,
        grid=(g,),
        in_specs=[
            pl.BlockSpec(memory_space=pltpu.SMEM),
            pl.BlockSpec(memory_space=pltpu.SMEM),
            pl.BlockSpec((4, c, 128), lambda i: (0, i, 0)),
        ],
        out_specs=pl.BlockSpec((c, 128), lambda i: (i, 0)),
        out_shape=jax.ShapeDtypeStruct((rows, 128), jnp.float32),
    )(we, be, ea_t)


_RB = 8


def _bmm_body(x_ref, w_ref, o_ref):
    for r in range(_RB):
        o_ref[r] = jnp.dot(x_ref[r], w_ref[r],
                           preferred_element_type=jnp.float32)


def _bmm(xt, w3):
    _, _, din = xt.shape
    dout = w3.shape[2]
    return pl.pallas_call(
        _bmm_body,
        grid=(R // _RB,),
        in_specs=[
            pl.BlockSpec((_RB, B, din), lambda r: (r, 0, 0)),
            pl.BlockSpec((_RB, din, dout), lambda r: (r, 0, 0)),
        ],
        out_specs=pl.BlockSpec((_RB, B, dout), lambda r: (r, 0, 0)),
        out_shape=jax.ShapeDtypeStruct((R, B, dout), jnp.float32),
    )(xt, w3)


def _finalize(num, aux, xt, bias):
    den = aux[:, :, 0]
    noloop = (aux[:, :, 1] == 0.0).astype(jnp.float32)
    return ((num + noloop[:, :, None] * xt)
            / jnp.maximum(den + noloop, 1.0)[:, :, None]
            + bias[None, None, :])


_GB = 6


def _rank_keep(score, elig, k):
    ii = lax.broadcasted_iota(jnp.int32, (_GB, R, R), 1)
    jj = lax.broadcasted_iota(jnp.int32, (_GB, R, R), 2)
    sj = score[:, None, :]
    si = score[:, :, None]
    above = (sj > si) | ((sj == si) & (jj < ii))
    if elig is not None:
        above = above & elig[:, None, :]
    rank = jnp.sum(above.astype(jnp.int32), axis=2)
    keep = rank < k
    if elig is not None:
        keep = keep & elig
    return keep


def _pool1_body(num0, num1, aux0, aux1, xt, bc1, wp1, keep_o, keepr_o, xp_o,
                x1_o):
    num = num0[...] + num1[...]
    aux = aux0[...] + aux1[...]
    h = _finalize(num, aux, xt[...], bc1[...])
    w = wp1[...]
    nrm = lax.rsqrt(jnp.sum(w * w))
    score = jax.nn.sigmoid(jnp.sum(h * w[None, None, :], axis=2) * nrm)
    keep = _rank_keep(score, None, K1)
    kf = keep.astype(jnp.float32)
    xp = h * (score * kf)[:, :, None]
    x1max = jnp.max(jnp.where(keep[:, :, None], xp, -jnp.inf), axis=1)
    x1mean = jnp.sum(xp, axis=1) * (1.0 / K1)
    keep_o[...] = kf[:, None, :]
    keepr_o[...] = kf[:, :, None] * jnp.ones((1, 1, 16), jnp.float32)
    xp_o[...] = xp
    x1_o[...] = jnp.concatenate([x1max, x1mean], axis=1)[:, None, :]


def _pool1(num0, num1, aux0, aux1, xt, bc1, wp1):
    return pl.pallas_call(
        _pool1_body,
        grid=(B // _GB,),
        in_specs=[
            pl.BlockSpec((_GB, R, DIM1), lambda b: (b, 0, 0)),
            pl.BlockSpec((_GB, R, DIM1), lambda b: (b, 0, 0)),
            pl.BlockSpec((_GB, R, 16), lambda b: (b, 0, 0)),
            pl.BlockSpec((_GB, R, 16), lambda b: (b, 0, 0)),
            pl.BlockSpec((_GB, R, DIM1), lambda b: (b, 0, 0)),
            pl.BlockSpec((DIM1,), lambda b: (0,)),
            pl.BlockSpec((DIM1,), lambda b: (0,)),
        ],
        out_specs=[
            pl.BlockSpec((_GB, 1, R), lambda b: (b, 0, 0)),
            pl.BlockSpec((_GB, R, 16), lambda b: (b, 0, 0)),
            pl.BlockSpec((_GB, R, DIM1), lambda b: (b, 0, 0)),
            pl.BlockSpec((_GB, 1, 2 * DIM1), lambda b: (b, 0, 0)),
        ],
        out_shape=[
            jax.ShapeDtypeStruct((B, 1, R), jnp.float32),
            jax.ShapeDtypeStruct((B, R, 16), jnp.float32),
            jax.ShapeDtypeStruct((B, R, DIM1), jnp.float32),
            jax.ShapeDtypeStruct((B, 1, 2 * DIM1), jnp.float32),
        ],
    )(num0, num1, aux0, aux1, xt, bc1, wp1)


def _pool2_body(num0, num1, aux0, aux1, xt, keep1, bc2, wp2, x2_o):
    num = num0[...] + num1[...]
    aux = aux0[...] + aux1[...]
    h = _finalize(num, aux, xt[...], bc2[...])
    w = wp2[...]
    nrm = lax.rsqrt(jnp.sum(w * w))
    score = jax.nn.sigmoid(jnp.sum(h * w[None, None, :], axis=2) * nrm)
    elig = keep1[:, 0, :] > 0.5
    keep = _rank_keep(score, elig, K2)
    kf = keep.astype(jnp.float32)
    xq = h * (score * kf)[:, :, None]
    x2max = jnp.max(jnp.where(keep[:, :, None], xq, -jnp.inf), axis=1)
    x2mean = jnp.sum(xq, axis=1) * (1.0 / K2)
    x2_o[...] = jnp.concatenate([x2max, x2mean], axis=1)[:, None, :]


def _pool2(num0, num1, aux0, aux1, xt, keep1, bc2, wp2):
    return pl.pallas_call(
        _pool2_body,
        grid=(B // _GB,),
        in_specs=[
            pl.BlockSpec((_GB, R, DIM2), lambda b: (b, 0, 0)),
            pl.BlockSpec((_GB, R, DIM2), lambda b: (b, 0, 0)),
            pl.BlockSpec((_GB, R, 16), lambda b: (b, 0, 0)),
            pl.BlockSpec((_GB, R, 16), lambda b: (b, 0, 0)),
            pl.BlockSpec((_GB, R, DIM2), lambda b: (b, 0, 0)),
            pl.BlockSpec((_GB, 1, R), lambda b: (b, 0, 0)),
            pl.BlockSpec((DIM2,), lambda b: (0,)),
            pl.BlockSpec((DIM2,), lambda b: (0,)),
        ],
        out_specs=pl.BlockSpec((_GB, 1, 2 * DIM2), lambda b: (b, 0, 0)),
        out_shape=jax.ShapeDtypeStruct((B, 1, 2 * DIM2), jnp.float32),
    )(num0, num1, aux0, aux1, xt, keep1, bc2, wp2)


_LN_SCALE = float(1.0 / (1.0 + 1e-5) ** 0.5)


def _head_body(x1, x2, wfc1, bfc1, g1, b1, wfc2, bfc2, g2, b2, wfc3, bfc3,
               out):
    z = jnp.concatenate([x1[...], x2[...]], axis=1)
    z = jnp.maximum(jnp.dot(z, wfc1[...], preferred_element_type=jnp.float32)
                    + bfc1[...][None, :], 0.0)
    z = z * (g1[...] * _LN_SCALE)[None, :] + b1[...][None, :]
    z = jnp.maximum(jnp.dot(z, wfc2[...], preferred_element_type=jnp.float32)
                    + bfc2[...][None, :], 0.0)
    z = z * (g2[...] * _LN_SCALE)[None, :] + b2[...][None, :]
    o = (jnp.dot(z, wfc3[...], preferred_element_type=jnp.float32)
         + bfc3[...][None, :])
    m = jnp.max(o, axis=1, keepdims=True)
    lse = jnp.log(jnp.sum(jnp.exp(o - m), axis=1, keepdims=True)) + m
    out[...] = o - lse


def _head(x1, x2, wfc1, bfc1, g1, b1, wfc2, bfc2, g2, b2, wfc3, bfc3):
    return pl.pallas_call(
        _head_body,
        out_shape=jax.ShapeDtypeStruct((B, 2), jnp.float32),
    )(x1, x2, wfc1, bfc1, g1, b1, wfc2, bfc2, g2, b2, wfc3, bfc3)


_GATHER_DNUMS = lax.GatherDimensionNumbers(
    offset_dims=(), collapsed_slice_dims=(0,), start_index_map=(0,))


def _splat(v, e):
    idx = jnp.full((16, 1), e, jnp.int32)
    return lax.gather(v, idx, _GATHER_DNUMS, (1,),
                      mode=lax.GatherScatterMode.PROMISE_IN_BOUNDS)


def _compute_chunk(pk_v, ew_v, rows_v, krs_v, krd_v, msg_v, aux_v,
                   lane, is0, is1, zeros16):
    for g in range(CHUNK // 16):
        sv = pk_v[0, pl.ds(g * 16, 16)]
        dv = pk_v[1, pl.ds(g * 16, 16)]
        ev = ew_v[pl.ds(g * 16, 16)]
        eqf = jnp.where(sv == dv, 1.0, 0.0)
        for e in range(16):
            i = g * 16 + e
            vals = krs_v[i, :] * krd_v[i, :]
            scs = _splat(ev, e) * vals
            lps = _splat(eqf, e) * vals
            msg_v[i, pl.ds(0, 16)] = rows_v[i, pl.ds(0, 16)] * scs
            msg_v[i, pl.ds(16, 16)] = rows_v[i, pl.ds(16, 16)] * scs
            aux_v[i, :] = jnp.where(is0, vals,
                                    jnp.where(is1, lps, zeros16))


def _edge_body(xt_hbm, epk_hbm, ew_hbm, keep_hbm, znum_hbm, zaux_hbm,
               num_out, aux_out,
               num_sh, aux_sh,
               pk_a, ew_a, src_a, dst_a, sdst_a, rows_a, krs_a, krd_a,
               msg_a, aux_a,
               pk_b, ew_b, src_b, dst_b, sdst_b, rows_b, krs_b, krd_b,
               msg_b, aux_b,
               gsem_a, ssem_a, gsem_b, ssem_b):
    c = lax.axis_index("c")
    s = lax.axis_index("s")
    wid = s * 2 + c
    sl = pl.ds(s * NPT, NPT)
    pltpu.sync_copy(znum_hbm.at[sl], num_sh.at[sl])
    pltpu.sync_copy(zaux_hbm.at[sl], aux_sh.at[sl])
    plsc.subcore_barrier()

    base_w = wid * EPW
    lane = lax.iota(jnp.int32, 16)
    is0 = (lane == 0)
    is1 = (lane == 1)
    zeros16 = jnp.zeros((16,), jnp.float32)

    def load_idx(ci, pk_v, ew_v, src_v, dst_v):
        base = jnp.minimum(base_w + ci * CHUNK, E - CHUNK)
        pltpu.sync_copy(epk_hbm.at[:, pl.ds(base, CHUNK)], pk_v)
        pltpu.sync_copy(ew_hbm.at[pl.ds(base, CHUNK)], ew_v)
        src_v[...] = pk_v[0, :]
        dst_v[...] = pk_v[1, :]

    def start_gathers(src_v, dst_v, rows_v, krs_v, krd_v, gsem):
        pltpu.async_copy(xt_hbm.at[src_v], rows_v, gsem)
        pltpu.async_copy(keep_hbm.at[src_v], krs_v, gsem)
        pltpu.async_copy(keep_hbm.at[dst_v], krd_v, gsem)

    def wait_gathers(src_v, dst_v, rows_v, krs_v, krd_v, gsem):
        pltpu.make_async_copy(xt_hbm.at[src_v], rows_v, gsem).wait()
        pltpu.make_async_copy(keep_hbm.at[src_v], krs_v, gsem).wait()
        pltpu.make_async_copy(keep_hbm.at[dst_v], krd_v, gsem).wait()

    def wait_scatters(sdst_v, msg_v, aux_v, ssem):
        pltpu.make_async_copy(msg_v, num_sh.at[sdst_v], ssem).wait()
        pltpu.make_async_copy(aux_v, aux_sh.at[sdst_v], ssem).wait()

    load_idx(0, pk_a, ew_a, src_a, dst_a)
    start_gathers(src_a, dst_a, rows_a, krs_a, krd_a, gsem_a)
    load_idx(1, pk_b, ew_b, src_b, dst_b)
    start_gathers(src_b, dst_b, rows_b, krs_b, krd_b, gsem_b)

    def half(j, cnext, pk_v, ew_v, src_v, dst_v, sdst_v, rows_v, krs_v,
             krd_v, msg_v, aux_v, gsem, ssem):
        @pl.when(j > 0)
        def _():
            wait_scatters(sdst_v, msg_v, aux_v, ssem)
        wait_gathers(src_v, dst_v, rows_v, krs_v, krd_v, gsem)
        _compute_chunk(pk_v, ew_v, rows_v, krs_v, krd_v, msg_v, aux_v,
                       lane, is0, is1, zeros16)
        for g in range(CHUNK // 16):
            sdst_v[pl.ds(g * 16, 16)] = dst_v[pl.ds(g * 16, 16)]
        pltpu.async_copy(msg_v, num_sh.at[sdst_v], ssem, add=True)
        pltpu.async_copy(aux_v, aux_sh.at[sdst_v], ssem, add=True)
        load_idx(cnext, pk_v, ew_v, src_v, dst_v)
        start_gathers(src_v, dst_v, rows_v, krs_v, krd_v, gsem)

    def pair_body(j, carry):
        half(j, 2 * j + 2, pk_a, ew_a, src_a, dst_a, sdst_a, rows_a,
             krs_a, krd_a, msg_a, aux_a, gsem_a, ssem_a)
        half(j, 2 * j + 3, pk_b, ew_b, src_b, dst_b, sdst_b, rows_b,
             krs_b, krd_b, msg_b, aux_b, gsem_b, ssem_b)
        return carry

    lax.fori_loop(0, NCHUNK // 2, pair_body, 0)
    wait_scatters(sdst_a, msg_a, aux_a, ssem_a)
    wait_scatters(sdst_b, msg_b, aux_b, ssem_b)
    wait_gathers(src_a, dst_a, rows_a, krs_a, krd_a, gsem_a)
    wait_gathers(src_b, dst_b, rows_b, krs_b, krd_b, gsem_b)
    plsc.subcore_barrier()
    pltpu.sync_copy(num_sh.at[sl], num_out.at[c, sl])
    pltpu.sync_copy(aux_sh.at[sl], aux_out.at[c, sl])


@functools.cache
def _edge_sc():
    return pl.kernel(
        _edge_body,
        out_type=[
            jax.ShapeDtypeStruct((2, N, DIM1), jnp.float32),
            jax.ShapeDtypeStruct((2, N, 16), jnp.float32),
        ],
        mesh=plsc.VectorSubcoreMesh(core_axis_name="c",
                                    subcore_axis_name="s"),
        compiler_params=pltpu.CompilerParams(use_tc_tiling_on_sc=False),
        scratch_types=[
            pltpu.VMEM_SHARED((N, DIM1), jnp.float32),
            pltpu.VMEM_SHARED((N, 16), jnp.float32),
        ] + 2 * [
            pltpu.VMEM((2, CHUNK), jnp.int32),
            pltpu.VMEM((CHUNK,), jnp.float32),
            pltpu.VMEM((CHUNK,), jnp.int32),
            pltpu.VMEM((CHUNK,), jnp.int32),
            pltpu.VMEM((CHUNK,), jnp.int32),
            pltpu.VMEM((CHUNK, DIM1), jnp.float32),
            pltpu.VMEM((CHUNK, 16), jnp.float32),
            pltpu.VMEM((CHUNK, 16), jnp.float32),
            pltpu.VMEM((CHUNK, DIM1), jnp.float32),
            pltpu.VMEM((CHUNK, 16), jnp.float32),
        ] + [
            pltpu.SemaphoreType.DMA,
            pltpu.SemaphoreType.DMA,
            pltpu.SemaphoreType.DMA,
            pltpu.SemaphoreType.DMA,
        ],
    )


def _edge_pass(xt, edge_pack, ew, keep_rep):
    znum = jnp.zeros((N, DIM1), jnp.float32)
    zaux = jnp.zeros((N, 16), jnp.float32)
    num, aux = _edge_sc()(xt, edge_pack, ew, keep_rep, znum, zaux)
    return num, aux


def kernel(x, edge_attr, We, be, Wn1a, Wn1b, bn1b, bc1, wp1, Wn2a, Wn2b,
           bn2b, bc2, wp2, Wfc1, bfc1, g1, b1, Wfc2, bfc2, g2, b2, Wfc3,
           bfc3, edge_index, y):
    edge_pack = edge_index

    w1flat, w2flat = _wgen(Wn1a, Wn1b, bn1b, Wn2a, Wn2b, bn2b)
    w1 = w1flat.reshape(R, R, DIM1)
    w2 = w2flat.reshape(R, DIM1, DIM2)

    ea_t = edge_attr.T.reshape(4, E // 128, 128)
    ew = _ew(We.reshape(4), be, ea_t).reshape(E)

    xt1 = (_bmm(x.reshape(B, R, R).transpose(1, 0, 2), w1)
           .transpose(1, 0, 2).reshape(N, DIM1))

    ones = jnp.ones((N, 16), jnp.float32)
    num1, aux1 = _edge_pass(xt1, edge_pack, ew, ones)

    xt1g = xt1.reshape(B, R, DIM1)
    keep1, keep1r, xp, x1 = _pool1(num1[0].reshape(B, R, DIM1),
                           num1[1].reshape(B, R, DIM1),
                           aux1[0].reshape(B, R, 16),
                           aux1[1].reshape(B, R, 16),
                           xt1g, bc1, wp1)

    xt2 = (_bmm(xp.transpose(1, 0, 2), w2)
           .transpose(1, 0, 2).reshape(N, DIM2))

    num2, aux2 = _edge_pass(xt2, edge_pack, ew, keep1r.reshape(N, 16))

    x2 = _pool2(num2[0].reshape(B, R, DIM2),
                num2[1].reshape(B, R, DIM2),
                aux2[0].reshape(B, R, 16),
                aux2[1].reshape(B, R, 16),
                xt2.reshape(B, R, DIM2), keep1, bc2, wp2)

    return _head(x1.reshape(B, 2 * DIM1), x2.reshape(B, 2 * DIM2),
                 Wfc1, bfc1, g1, b1, Wfc2, bfc2, g2, b2, Wfc3, bfc3)
